# baseline (speedup 1.0000x reference)
import functools
import jax
import jax.numpy as jnp
from jax.experimental import pallas as pl
from jax.experimental.pallas import tpu as pltpu

HEADS = 8


def _mm_body(a_ref, b_ref, o_ref):
    o_ref[...] = jnp.dot(a_ref[...], b_ref[...],
                         preferred_element_type=jnp.float32)


def _matmul(a, b, bm=1024):
    m, k = a.shape
    _, f = b.shape
    mp = ((m + bm - 1) // bm) * bm
    if mp != m:
        a = jnp.pad(a, ((0, mp - m), (0, 0)))
    out = pl.pallas_call(
        _mm_body,
        grid=(mp // bm,),
        in_specs=[
            pl.BlockSpec((bm, k), lambda i: (i, 0)),
            pl.BlockSpec((k, f), lambda i: (0, 0)),
        ],
        out_specs=pl.BlockSpec((bm, f), lambda i: (i, 0)),
        out_shape=jax.ShapeDtypeStruct((mp, f), jnp.float32),
    )(a, b)
    return out[:m] if mp != m else out


def _gat_layer(h, src, dst, W, a_s, a_d, bias, out_ch, concat):
    n = h.shape[0]
    xW = _matmul(h, W).reshape(n, HEADS, out_ch)
    alpha_src = jnp.einsum('nhc,hc->nh', xW, a_s)
    alpha_dst = jnp.einsum('nhc,hc->nh', xW, a_d)
    e = alpha_src[src] + alpha_dst[dst]
    e = jnp.where(e > 0, e, 0.2 * e)
    m = jnp.max(e, axis=0)
    ex = jnp.exp(e - m[None, :])
    denom = jax.ops.segment_sum(ex, dst, num_segments=n)
    num = jax.ops.segment_sum(xW[src] * ex[:, :, None], dst, num_segments=n)
    out = num / (denom[:, :, None] + 1e-16)
    if concat:
        out = out.reshape(n, HEADS * out_ch)
    else:
        out = out.mean(axis=1)
    return out + bias


def _bn_elu(x, g, b, eps=1e-5):
    m = x.mean(0)
    v = x.var(0)
    y = g * (x - m) / jnp.sqrt(v + eps) + b
    return jax.nn.elu(y)


def kernel(x, edge_index, W0, as0, ad0, b0, g0, be0, W1, as1, ad1, b1, g1,
           be1, W2, as2, ad2, b2, g2, be2, Wc, bc):
    n = x.shape[0]
    loop = jnp.arange(n, dtype=edge_index.dtype)
    src = jnp.concatenate([edge_index[0], loop])
    dst = jnp.concatenate([edge_index[1], loop])
    h = _gat_layer(x, src, dst, W0, as0, ad0, b0, 32, True)
    h = _bn_elu(h, g0, be0)
    h_in = h
    h = _gat_layer(h, src, dst, W1, as1, ad1, b1, 32, True)
    h = _bn_elu(h, g1, be1) + h_in
    h_in = h
    h = _gat_layer(h, src, dst, W2, as2, ad2, b2, 256, False)
    h = _bn_elu(h, g2, be2) + h_in
    return _matmul(h, Wc) + bc
